# baseline (speedup 1.0000x reference)
import functools

import jax
import jax.numpy as jnp
import numpy as np
from jax import lax
from jax.experimental import pallas as pl
from jax.experimental.pallas import tpu as pltpu
from jax.experimental.pallas import tpu_sc as plsc


_N = 2048
_NC, _NS = 2, 16
_NW = _NC * _NS


def _nnd_sc(input1, input2):
    bsz, n, _ = input1.shape
    t1 = jnp.transpose(input1, (0, 2, 1))
    t2 = jnp.transpose(input2, (0, 2, 1))
    q = jnp.stack([t1, t2], axis=1).reshape(2 * bsz, 3, n)
    r = jnp.stack([t2, t1], axis=1).reshape(2 * bsz, 3, n)
    ndir = 2 * bsz
    wpd = max(_NW // ndir, 1)
    qpw = n // wpd
    ngrp = qpw // 16

    mesh = plsc.VectorSubcoreMesh(
        core_axis_name="c", subcore_axis_name="s",
        num_cores=_NC, num_subcores=_NS,
    )

    @functools.partial(
        pl.kernel,
        out_type=jax.ShapeDtypeStruct((ndir * n,), jnp.float32),
        mesh=mesh,
        scratch_types=[
            pltpu.VMEM((n,), jnp.float32),
            pltpu.VMEM((n,), jnp.float32),
            pltpu.VMEM((n,), jnp.float32),
            pltpu.VMEM((qpw,), jnp.float32),
            pltpu.VMEM((qpw,), jnp.float32),
            pltpu.VMEM((qpw,), jnp.float32),
            pltpu.VMEM((qpw,), jnp.float32),
        ],
    )
    def sc_kernel(q_hbm, r_hbm, out_hbm, rx, ry, rz, qx, qy, qz, ob):
        wid = lax.axis_index("s") * _NC + lax.axis_index("c")
        d = wid // wpd
        qb = (wid % wpd) * qpw
        rbase = d * 3 * n
        qbase = d * 3 * n + qb
        pltpu.sync_copy(r_hbm.at[pl.ds(rbase, n)], rx)
        pltpu.sync_copy(r_hbm.at[pl.ds(rbase + n, n)], ry)
        pltpu.sync_copy(r_hbm.at[pl.ds(rbase + 2 * n, n)], rz)
        pltpu.sync_copy(q_hbm.at[pl.ds(qbase, qpw)], qx)
        pltpu.sync_copy(q_hbm.at[pl.ds(qbase + n, qpw)], qy)
        pltpu.sync_copy(q_hbm.at[pl.ds(qbase + 2 * n, qpw)], qz)

        def group_body(g, _):
            base = g * 16
            qxv = qx[pl.ds(base, 16)]
            qyv = qy[pl.ds(base, 16)]
            qzv = qz[pl.ds(base, 16)]
            m0 = jnp.full((16,), 3.4e38, jnp.float32)

            def ref_body(c, m):
                rbase = c * 16
                rxv = rx[pl.ds(rbase, 16)]
                ryv = ry[pl.ds(rbase, 16)]
                rzv = rz[pl.ds(rbase, 16)]
                for k in range(16):
                    dx = qxv - jnp.full((16,), rxv[k])
                    dy = qyv - jnp.full((16,), ryv[k])
                    dz = qzv - jnp.full((16,), rzv[k])
                    dd = dx * dx + dy * dy + dz * dz
                    m = jnp.minimum(m, dd)
                return m

            m = lax.fori_loop(0, n // 16, ref_body, m0)
            ob[pl.ds(base, 16)] = m
            return 0

        lax.fori_loop(0, ngrp, group_body, 0)
        pltpu.sync_copy(ob, out_hbm.at[pl.ds(d * n + qb, qpw)])

    out = sc_kernel(q.reshape(-1), r.reshape(-1)).reshape(ndir, n)
    return out[0::2], out[1::2]


def _hi_lo(x):
    hi = x.astype(jnp.bfloat16)
    lo = (x - hi.astype(jnp.float32)).astype(jnp.bfloat16)
    return hi, lo


def _nnd_batch_kernel(p1_ref, p2_ref, d1_ref, d2_ref):
    p1 = p1_ref[0]
    p2 = p2_ref[0]
    n1 = jnp.sum(p1 * p1, axis=1, keepdims=True)
    n2 = jnp.sum(p2 * p2, axis=1, keepdims=True)
    b2 = -2.0 * p2
    p1h, p1l = _hi_lo(p1)
    b2h, b2l = _hi_lo(b2)
    n1h, n1l = _hi_lo(n1)
    n2h, n2l = _hi_lo(n2)
    one = jnp.ones_like(n1h)
    a = jnp.concatenate([p1h, p1h, p1l, n1h, n1l, one, one], axis=1)
    b = jnp.concatenate([b2h, b2l, b2h, one, one, n2h, n2l], axis=1)
    d = jax.lax.dot_general(
        a, b, (((1,), (1,)), ((), ())),
        preferred_element_type=jnp.float32,
    )
    d1_ref[0, 0] = jnp.min(d, axis=1)
    d2_ref[0, 0] = jnp.min(d, axis=0)


def _nnd_pallas(input1, input2):
    bsz, n, _ = input1.shape
    grid = (bsz,)
    out_shape = (
        jax.ShapeDtypeStruct((bsz, 1, n), jnp.float32),
        jax.ShapeDtypeStruct((bsz, 1, n), jnp.float32),
    )
    d1, d2 = pl.pallas_call(
        _nnd_batch_kernel,
        grid=grid,
        in_specs=[
            pl.BlockSpec((1, n, 3), lambda b: (b, 0, 0)),
            pl.BlockSpec((1, n, 3), lambda b: (b, 0, 0)),
        ],
        out_specs=(
            pl.BlockSpec((1, 1, n), lambda b: (b, 0, 0)),
            pl.BlockSpec((1, 1, n), lambda b: (b, 0, 0)),
        ),
        out_shape=out_shape,
    )(input1, input2)
    return d1.reshape(bsz, n), d2.reshape(bsz, n)


def kernel(input1, input2):
    return _nnd_sc(input1, input2)


# speedup vs baseline: 3.9670x; 3.9670x over previous
import functools

import jax
import jax.numpy as jnp
import numpy as np
from jax import lax
from jax.experimental import pallas as pl
from jax.experimental.pallas import tpu as pltpu
from jax.experimental.pallas import tpu_sc as plsc


_N = 2048
_NC, _NS = 2, 16
_NW = _NC * _NS


def _nnd_sc(input1, input2):
    bsz, n, _ = input1.shape
    t1 = jnp.transpose(input1, (0, 2, 1))
    t2 = jnp.transpose(input2, (0, 2, 1))
    q = jnp.stack([t1, t2], axis=1).reshape(2 * bsz, 3, n)
    r = jnp.stack([t2, t1], axis=1).reshape(2 * bsz, 3, n)
    ndir = 2 * bsz
    wpd = max(_NW // ndir, 1)
    qpw = n // wpd
    ngrp = qpw // 16

    mesh = plsc.VectorSubcoreMesh(
        core_axis_name="c", subcore_axis_name="s",
        num_cores=_NC, num_subcores=_NS,
    )

    @functools.partial(
        pl.kernel,
        out_type=jax.ShapeDtypeStruct((ndir * n,), jnp.float32),
        mesh=mesh,
        scratch_types=[
            pltpu.VMEM((n,), jnp.float32),
            pltpu.VMEM((n,), jnp.float32),
            pltpu.VMEM((n,), jnp.float32),
            pltpu.VMEM((qpw,), jnp.float32),
            pltpu.VMEM((qpw,), jnp.float32),
            pltpu.VMEM((qpw,), jnp.float32),
            pltpu.VMEM((qpw,), jnp.float32),
        ],
    )
    def sc_kernel(q_hbm, r_hbm, out_hbm, rx, ry, rz, qx, qy, qz, ob):
        wid = lax.axis_index("s") * _NC + lax.axis_index("c")
        d = wid // wpd
        qb = (wid % wpd) * qpw
        rbase = d * 3 * n
        qbase = d * 3 * n + qb
        pltpu.sync_copy(r_hbm.at[pl.ds(rbase, n)], rx)
        pltpu.sync_copy(r_hbm.at[pl.ds(rbase + n, n)], ry)
        pltpu.sync_copy(r_hbm.at[pl.ds(rbase + 2 * n, n)], rz)
        pltpu.sync_copy(q_hbm.at[pl.ds(qbase, qpw)], qx)
        pltpu.sync_copy(q_hbm.at[pl.ds(qbase + n, qpw)], qy)
        pltpu.sync_copy(q_hbm.at[pl.ds(qbase + 2 * n, qpw)], qz)

        def group_body(g, _):
            base = g * 16
            qxv = qx[pl.ds(base, 16)]
            qyv = qy[pl.ds(base, 16)]
            qzv = qz[pl.ds(base, 16)]
            m0 = jnp.full((16,), 3.4e38, jnp.float32)

            def ref_body(c, m):
                rbase = c * 16
                rxv = rx[pl.ds(rbase, 16)]
                ryv = ry[pl.ds(rbase, 16)]
                rzv = rz[pl.ds(rbase, 16)]
                for k in range(16):
                    dx = qxv - jnp.full((16,), rxv[k])
                    dy = qyv - jnp.full((16,), ryv[k])
                    dz = qzv - jnp.full((16,), rzv[k])
                    dd = dx * dx + dy * dy + dz * dz
                    m = jnp.minimum(m, dd)
                return m

            m = lax.fori_loop(0, n // 16, ref_body, m0)
            ob[pl.ds(base, 16)] = m
            return 0

        lax.fori_loop(0, ngrp, group_body, 0)
        pltpu.sync_copy(ob, out_hbm.at[pl.ds(d * n + qb, qpw)])

    out = sc_kernel(q.reshape(-1), r.reshape(-1)).reshape(ndir, n)
    return out[0::2], out[1::2]


def _hi_lo(x):
    hi = x.astype(jnp.bfloat16)
    lo = (x - hi.astype(jnp.float32)).astype(jnp.bfloat16)
    return hi, lo


def _nnd_batch_kernel(p1_ref, p2_ref, d1_ref, d2_ref):
    p1 = p1_ref[0]
    p2 = p2_ref[0]
    n1 = jnp.sum(p1 * p1, axis=1, keepdims=True)
    n2 = jnp.sum(p2 * p2, axis=1, keepdims=True)
    b2 = -2.0 * p2
    p1h, p1l = _hi_lo(p1)
    b2h, b2l = _hi_lo(b2)
    n1h, n1l = _hi_lo(n1)
    n2h, n2l = _hi_lo(n2)
    one = jnp.ones_like(n1h)
    a = jnp.concatenate([p1h, p1h, p1l, n1h, n1l, one, one], axis=1)
    b = jnp.concatenate([b2h, b2l, b2h, one, one, n2h, n2l], axis=1)
    d = jax.lax.dot_general(
        a, b, (((1,), (1,)), ((), ())),
        preferred_element_type=jnp.float32,
    )
    d1_ref[0, 0] = jnp.min(d, axis=1)
    d2_ref[0, 0] = jnp.min(d, axis=0)


def _nnd_pallas(input1, input2):
    bsz, n, _ = input1.shape
    grid = (bsz,)
    out_shape = (
        jax.ShapeDtypeStruct((bsz, 1, n), jnp.float32),
        jax.ShapeDtypeStruct((bsz, 1, n), jnp.float32),
    )
    d1, d2 = pl.pallas_call(
        _nnd_batch_kernel,
        grid=grid,
        in_specs=[
            pl.BlockSpec((1, n, 3), lambda b: (b, 0, 0)),
            pl.BlockSpec((1, n, 3), lambda b: (b, 0, 0)),
        ],
        out_specs=(
            pl.BlockSpec((1, 1, n), lambda b: (b, 0, 0)),
            pl.BlockSpec((1, 1, n), lambda b: (b, 0, 0)),
        ),
        out_shape=out_shape,
    )(input1, input2)
    return d1.reshape(bsz, n), d2.reshape(bsz, n)


def kernel(input1, input2):
    d1_sc, d2_sc = _nnd_sc(input1[:1], input2[:1])
    d1_tc, d2_tc = _nnd_pallas(input1[1:], input2[1:])
    return (jnp.concatenate([d1_sc, d1_tc], axis=0),
            jnp.concatenate([d2_sc, d2_tc], axis=0))


# speedup vs baseline: 6.8640x; 1.7303x over previous
import functools

import jax
import jax.numpy as jnp
import numpy as np
from jax import lax
from jax.experimental import pallas as pl
from jax.experimental.pallas import tpu as pltpu
from jax.experimental.pallas import tpu_sc as plsc


_N = 2048
_NC, _NS = 2, 16
_NW = _NC * _NS


def _nnd_sc(input1, input2):
    bsz, n, _ = input1.shape
    t1 = jnp.transpose(input1, (0, 2, 1))
    t2 = jnp.transpose(input2, (0, 2, 1))
    q = jnp.stack([t1, t2], axis=1).reshape(2 * bsz, 3, n)
    r = jnp.stack([t2, t1], axis=1).reshape(2 * bsz, 3, n)
    ndir = 2 * bsz
    wpd = max(_NW // ndir, 1)
    qpw = n // wpd
    ngrp = qpw // 16

    mesh = plsc.VectorSubcoreMesh(
        core_axis_name="c", subcore_axis_name="s",
        num_cores=_NC, num_subcores=_NS,
    )

    @functools.partial(
        pl.kernel,
        out_type=jax.ShapeDtypeStruct((ndir * n,), jnp.float32),
        mesh=mesh,
        scratch_types=[
            pltpu.VMEM((n,), jnp.float32),
            pltpu.VMEM((n,), jnp.float32),
            pltpu.VMEM((n,), jnp.float32),
            pltpu.VMEM((qpw,), jnp.float32),
            pltpu.VMEM((qpw,), jnp.float32),
            pltpu.VMEM((qpw,), jnp.float32),
            pltpu.VMEM((qpw,), jnp.float32),
        ],
    )
    def sc_kernel(q_hbm, r_hbm, out_hbm, rx, ry, rz, qx, qy, qz, ob):
        wid = lax.axis_index("s") * _NC + lax.axis_index("c")
        d = wid // wpd
        qb = (wid % wpd) * qpw
        rbase = d * 3 * n
        qbase = d * 3 * n + qb
        pltpu.sync_copy(r_hbm.at[pl.ds(rbase, n)], rx)
        pltpu.sync_copy(r_hbm.at[pl.ds(rbase + n, n)], ry)
        pltpu.sync_copy(r_hbm.at[pl.ds(rbase + 2 * n, n)], rz)
        pltpu.sync_copy(q_hbm.at[pl.ds(qbase, qpw)], qx)
        pltpu.sync_copy(q_hbm.at[pl.ds(qbase + n, qpw)], qy)
        pltpu.sync_copy(q_hbm.at[pl.ds(qbase + 2 * n, qpw)], qz)

        def group_body(g, _):
            base = g * 16
            qxv = qx[pl.ds(base, 16)]
            qyv = qy[pl.ds(base, 16)]
            qzv = qz[pl.ds(base, 16)]
            m0 = jnp.full((16,), 3.4e38, jnp.float32)

            def ref_body(c, m):
                rbase = c * 16
                rxv = rx[pl.ds(rbase, 16)]
                ryv = ry[pl.ds(rbase, 16)]
                rzv = rz[pl.ds(rbase, 16)]
                for k in range(16):
                    dx = qxv - jnp.full((16,), rxv[k])
                    dy = qyv - jnp.full((16,), ryv[k])
                    dz = qzv - jnp.full((16,), rzv[k])
                    dd = dx * dx + dy * dy + dz * dz
                    m = jnp.minimum(m, dd)
                return m

            m = lax.fori_loop(0, n // 16, ref_body, m0)
            ob[pl.ds(base, 16)] = m
            return 0

        lax.fori_loop(0, ngrp, group_body, 0)
        pltpu.sync_copy(ob, out_hbm.at[pl.ds(d * n + qb, qpw)])

    out = sc_kernel(q.reshape(-1), r.reshape(-1)).reshape(ndir, n)
    return out[0::2], out[1::2]


def _hi_lo(x):
    hi = x.astype(jnp.bfloat16)
    lo = (x - hi.astype(jnp.float32)).astype(jnp.bfloat16)
    return hi, lo


def _nnd_one(p1, p2, d1_out, d2_out):
    n1 = jnp.sum(p1 * p1, axis=0, keepdims=True)
    n2 = jnp.sum(p2 * p2, axis=0, keepdims=True)
    b2 = -2.0 * p2
    p1h, p1l = _hi_lo(p1)
    b2h, b2l = _hi_lo(b2)
    n1h, n1l = _hi_lo(n1)
    n2h, n2l = _hi_lo(n2)
    one = jnp.ones_like(n1h)
    a = jnp.concatenate([p1h, p1h, p1l, n1h, n1l, one, one], axis=0)
    b = jnp.concatenate([b2h, b2l, b2h, one, one, n2h, n2l], axis=0)
    d = jax.lax.dot_general(
        a, b, (((0,), (0,)), ((), ())),
        preferred_element_type=jnp.float32,
    )
    d1_out[...] = jnp.min(d, axis=1)
    d2_out[...] = jnp.min(d, axis=0)


def _nnd_batch_kernel(p1_ref, p2_ref, d1_ref, d2_ref):
    for b in range(p1_ref.shape[0]):
        _nnd_one(p1_ref[b], p2_ref[b], d1_ref.at[b, 0], d2_ref.at[b, 0])


def _nnd_pallas(input1, input2):
    bsz, n, _ = input1.shape
    t1 = jnp.transpose(input1, (0, 2, 1))
    t2 = jnp.transpose(input2, (0, 2, 1))
    bpb = 1
    grid = (bsz // bpb,)
    out_shape = (
        jax.ShapeDtypeStruct((bsz, 1, n), jnp.float32),
        jax.ShapeDtypeStruct((bsz, 1, n), jnp.float32),
    )
    d1, d2 = pl.pallas_call(
        _nnd_batch_kernel,
        grid=grid,
        in_specs=[
            pl.BlockSpec((bpb, 3, n), lambda b: (b, 0, 0)),
            pl.BlockSpec((bpb, 3, n), lambda b: (b, 0, 0)),
        ],
        out_specs=(
            pl.BlockSpec((bpb, 1, n), lambda b: (b, 0, 0)),
            pl.BlockSpec((bpb, 1, n), lambda b: (b, 0, 0)),
        ),
        out_shape=out_shape,
    )(t1, t2)
    return d1.reshape(bsz, n), d2.reshape(bsz, n)


def kernel(input1, input2):
    return _nnd_pallas(input1, input2)


# speedup vs baseline: 8.2093x; 1.1960x over previous
import functools

import jax
import jax.numpy as jnp
import numpy as np
from jax import lax
from jax.experimental import pallas as pl
from jax.experimental.pallas import tpu as pltpu
from jax.experimental.pallas import tpu_sc as plsc


_N = 2048
_NC, _NS = 2, 16
_NW = _NC * _NS


def _nnd_sc(input1, input2):
    bsz, n, _ = input1.shape
    t1 = jnp.transpose(input1, (0, 2, 1))
    t2 = jnp.transpose(input2, (0, 2, 1))
    q = jnp.stack([t1, t2], axis=1).reshape(2 * bsz, 3, n)
    r = jnp.stack([t2, t1], axis=1).reshape(2 * bsz, 3, n)
    ndir = 2 * bsz
    wpd = max(_NW // ndir, 1)
    qpw = n // wpd
    ngrp = qpw // 16

    mesh = plsc.VectorSubcoreMesh(
        core_axis_name="c", subcore_axis_name="s",
        num_cores=_NC, num_subcores=_NS,
    )

    @functools.partial(
        pl.kernel,
        out_type=jax.ShapeDtypeStruct((ndir * n,), jnp.float32),
        mesh=mesh,
        scratch_types=[
            pltpu.VMEM((n,), jnp.float32),
            pltpu.VMEM((n,), jnp.float32),
            pltpu.VMEM((n,), jnp.float32),
            pltpu.VMEM((qpw,), jnp.float32),
            pltpu.VMEM((qpw,), jnp.float32),
            pltpu.VMEM((qpw,), jnp.float32),
            pltpu.VMEM((qpw,), jnp.float32),
        ],
    )
    def sc_kernel(q_hbm, r_hbm, out_hbm, rx, ry, rz, qx, qy, qz, ob):
        wid = lax.axis_index("s") * _NC + lax.axis_index("c")
        d = wid // wpd
        qb = (wid % wpd) * qpw
        rbase = d * 3 * n
        qbase = d * 3 * n + qb
        pltpu.sync_copy(r_hbm.at[pl.ds(rbase, n)], rx)
        pltpu.sync_copy(r_hbm.at[pl.ds(rbase + n, n)], ry)
        pltpu.sync_copy(r_hbm.at[pl.ds(rbase + 2 * n, n)], rz)
        pltpu.sync_copy(q_hbm.at[pl.ds(qbase, qpw)], qx)
        pltpu.sync_copy(q_hbm.at[pl.ds(qbase + n, qpw)], qy)
        pltpu.sync_copy(q_hbm.at[pl.ds(qbase + 2 * n, qpw)], qz)

        def group_body(g, _):
            base = g * 16
            qxv = qx[pl.ds(base, 16)]
            qyv = qy[pl.ds(base, 16)]
            qzv = qz[pl.ds(base, 16)]
            m0 = jnp.full((16,), 3.4e38, jnp.float32)

            def ref_body(c, m):
                rbase = c * 16
                rxv = rx[pl.ds(rbase, 16)]
                ryv = ry[pl.ds(rbase, 16)]
                rzv = rz[pl.ds(rbase, 16)]
                for k in range(16):
                    dx = qxv - jnp.full((16,), rxv[k])
                    dy = qyv - jnp.full((16,), ryv[k])
                    dz = qzv - jnp.full((16,), rzv[k])
                    dd = dx * dx + dy * dy + dz * dz
                    m = jnp.minimum(m, dd)
                return m

            m = lax.fori_loop(0, n // 16, ref_body, m0)
            ob[pl.ds(base, 16)] = m
            return 0

        lax.fori_loop(0, ngrp, group_body, 0)
        pltpu.sync_copy(ob, out_hbm.at[pl.ds(d * n + qb, qpw)])

    out = sc_kernel(q.reshape(-1), r.reshape(-1)).reshape(ndir, n)
    return out[0::2], out[1::2]


def _hi_lo(x):
    hi = x.astype(jnp.bfloat16)
    lo = (x - hi.astype(jnp.float32)).astype(jnp.bfloat16)
    return hi, lo


def _nnd_one(p1, p2, d1_out, d2_out):
    n1 = jnp.sum(p1 * p1, axis=0, keepdims=True)
    n2 = jnp.sum(p2 * p2, axis=0, keepdims=True)
    b2 = -2.0 * p2
    p1h, p1l = _hi_lo(p1)
    b2h, b2l = _hi_lo(b2)
    n1h, n1l = _hi_lo(n1)
    n2h, n2l = _hi_lo(n2)
    one = jnp.ones_like(n1h)
    a = jnp.concatenate([p1h, p1h, p1l, n1h, n1l, one, one], axis=0)
    b = jnp.concatenate([b2h, b2l, b2h, one, one, n2h, n2l], axis=0)
    d = jax.lax.dot_general(
        a, b, (((0,), (0,)), ((), ())),
        preferred_element_type=jnp.float32,
    )
    d1_out[...] = jnp.min(d, axis=1)
    d2_out[...] = jnp.min(d, axis=0)


def _nnd_batch_kernel(p1_ref, p2_ref, d1_ref, d2_ref):
    for b in range(p1_ref.shape[0]):
        _nnd_one(p1_ref[b], p2_ref[b], d1_ref.at[b, 0], d2_ref.at[b, 0])


def _nnd_pallas(input1, input2):
    bsz, n, _ = input1.shape
    t1 = jnp.transpose(input1, (0, 2, 1))
    t2 = jnp.transpose(input2, (0, 2, 1))
    bpb = 2 if bsz % 2 == 0 else 1
    grid = (bsz // bpb,)
    out_shape = (
        jax.ShapeDtypeStruct((bsz, 1, n), jnp.float32),
        jax.ShapeDtypeStruct((bsz, 1, n), jnp.float32),
    )
    d1, d2 = pl.pallas_call(
        _nnd_batch_kernel,
        grid=grid,
        in_specs=[
            pl.BlockSpec((bpb, 3, n), lambda b: (b, 0, 0)),
            pl.BlockSpec((bpb, 3, n), lambda b: (b, 0, 0)),
        ],
        out_specs=(
            pl.BlockSpec((bpb, 1, n), lambda b: (b, 0, 0)),
            pl.BlockSpec((bpb, 1, n), lambda b: (b, 0, 0)),
        ),
        out_shape=out_shape,
    )(t1, t2)
    return d1.reshape(bsz, n), d2.reshape(bsz, n)


def kernel(input1, input2):
    return _nnd_pallas(input1, input2)


# speedup vs baseline: 9.0672x; 1.1045x over previous
import functools

import jax
import jax.numpy as jnp
import numpy as np
from jax import lax
from jax.experimental import pallas as pl
from jax.experimental.pallas import tpu as pltpu
from jax.experimental.pallas import tpu_sc as plsc


_N = 2048
_NC, _NS = 2, 16
_NW = _NC * _NS


def _nnd_sc(input1, input2):
    bsz, n, _ = input1.shape
    t1 = jnp.transpose(input1, (0, 2, 1))
    t2 = jnp.transpose(input2, (0, 2, 1))
    q = jnp.stack([t1, t2], axis=1).reshape(2 * bsz, 3, n)
    r = jnp.stack([t2, t1], axis=1).reshape(2 * bsz, 3, n)
    ndir = 2 * bsz
    wpd = max(_NW // ndir, 1)
    qpw = n // wpd
    ngrp = qpw // 16

    mesh = plsc.VectorSubcoreMesh(
        core_axis_name="c", subcore_axis_name="s",
        num_cores=_NC, num_subcores=_NS,
    )

    @functools.partial(
        pl.kernel,
        out_type=jax.ShapeDtypeStruct((ndir * n,), jnp.float32),
        mesh=mesh,
        scratch_types=[
            pltpu.VMEM((n,), jnp.float32),
            pltpu.VMEM((n,), jnp.float32),
            pltpu.VMEM((n,), jnp.float32),
            pltpu.VMEM((qpw,), jnp.float32),
            pltpu.VMEM((qpw,), jnp.float32),
            pltpu.VMEM((qpw,), jnp.float32),
            pltpu.VMEM((qpw,), jnp.float32),
        ],
    )
    def sc_kernel(q_hbm, r_hbm, out_hbm, rx, ry, rz, qx, qy, qz, ob):
        wid = lax.axis_index("s") * _NC + lax.axis_index("c")
        d = wid // wpd
        qb = (wid % wpd) * qpw
        rbase = d * 3 * n
        qbase = d * 3 * n + qb
        pltpu.sync_copy(r_hbm.at[pl.ds(rbase, n)], rx)
        pltpu.sync_copy(r_hbm.at[pl.ds(rbase + n, n)], ry)
        pltpu.sync_copy(r_hbm.at[pl.ds(rbase + 2 * n, n)], rz)
        pltpu.sync_copy(q_hbm.at[pl.ds(qbase, qpw)], qx)
        pltpu.sync_copy(q_hbm.at[pl.ds(qbase + n, qpw)], qy)
        pltpu.sync_copy(q_hbm.at[pl.ds(qbase + 2 * n, qpw)], qz)

        def group_body(g, _):
            base = g * 16
            qxv = qx[pl.ds(base, 16)]
            qyv = qy[pl.ds(base, 16)]
            qzv = qz[pl.ds(base, 16)]
            m0 = jnp.full((16,), 3.4e38, jnp.float32)

            def ref_body(c, m):
                rbase = c * 16
                rxv = rx[pl.ds(rbase, 16)]
                ryv = ry[pl.ds(rbase, 16)]
                rzv = rz[pl.ds(rbase, 16)]
                for k in range(16):
                    dx = qxv - jnp.full((16,), rxv[k])
                    dy = qyv - jnp.full((16,), ryv[k])
                    dz = qzv - jnp.full((16,), rzv[k])
                    dd = dx * dx + dy * dy + dz * dz
                    m = jnp.minimum(m, dd)
                return m

            m = lax.fori_loop(0, n // 16, ref_body, m0)
            ob[pl.ds(base, 16)] = m
            return 0

        lax.fori_loop(0, ngrp, group_body, 0)
        pltpu.sync_copy(ob, out_hbm.at[pl.ds(d * n + qb, qpw)])

    out = sc_kernel(q.reshape(-1), r.reshape(-1)).reshape(ndir, n)
    return out[0::2], out[1::2]


def _hi_lo(x):
    hi = x.astype(jnp.bfloat16)
    lo = (x - hi.astype(jnp.float32)).astype(jnp.bfloat16)
    return hi, lo


def _nnd_one(p1, p2, d1_out, d2_out):
    n1 = jnp.sum(p1 * p1, axis=0, keepdims=True)
    n2 = jnp.sum(p2 * p2, axis=0, keepdims=True)
    b2 = -2.0 * p2
    p1h, p1l = _hi_lo(p1)
    b2h, b2l = _hi_lo(b2)
    n1h, n1l = _hi_lo(n1)
    n2h, n2l = _hi_lo(n2)
    one = jnp.ones_like(n1h)
    a = jnp.concatenate([p1h, p1h, p1l, n1h, n1l, one, one], axis=0)
    b = jnp.concatenate([b2h, b2l, b2h, one, one, n2h, n2l], axis=0)
    n = p1.shape[1]
    nh = n // 2
    m1 = None
    for jh in range(2):
        d = jax.lax.dot_general(
            a, b[:, jh * nh:(jh + 1) * nh], (((0,), (0,)), ((), ())),
            preferred_element_type=jnp.float32,
        )
        mh = jnp.min(d, axis=1)
        m1 = mh if m1 is None else jnp.minimum(m1, mh)
        d2_out[pl.ds(jh * nh, nh)] = jnp.min(d, axis=0)
    d1_out[...] = m1


def _nnd_batch_kernel(p1_ref, p2_ref, d1_ref, d2_ref):
    for b in range(p1_ref.shape[0]):
        _nnd_one(p1_ref[b], p2_ref[b], d1_ref.at[b, 0], d2_ref.at[b, 0])


def _nnd_pallas(input1, input2):
    bsz, n, _ = input1.shape
    t1 = jnp.transpose(input1, (0, 2, 1))
    t2 = jnp.transpose(input2, (0, 2, 1))
    bpb = 4 if bsz % 4 == 0 else 1
    grid = (bsz // bpb,)
    out_shape = (
        jax.ShapeDtypeStruct((bsz, 1, n), jnp.float32),
        jax.ShapeDtypeStruct((bsz, 1, n), jnp.float32),
    )
    d1, d2 = pl.pallas_call(
        _nnd_batch_kernel,
        grid=grid,
        in_specs=[
            pl.BlockSpec((bpb, 3, n), lambda b: (b, 0, 0)),
            pl.BlockSpec((bpb, 3, n), lambda b: (b, 0, 0)),
        ],
        out_specs=(
            pl.BlockSpec((bpb, 1, n), lambda b: (b, 0, 0)),
            pl.BlockSpec((bpb, 1, n), lambda b: (b, 0, 0)),
        ),
        out_shape=out_shape,
    )(t1, t2)
    return d1.reshape(bsz, n), d2.reshape(bsz, n)


def kernel(input1, input2):
    return _nnd_pallas(input1, input2)


# speedup vs baseline: 9.6083x; 1.0597x over previous
import functools

import jax
import jax.numpy as jnp
import numpy as np
from jax import lax
from jax.experimental import pallas as pl
from jax.experimental.pallas import tpu as pltpu
from jax.experimental.pallas import tpu_sc as plsc


_N = 2048
_NC, _NS = 2, 16
_NW = _NC * _NS


def _nnd_sc(input1, input2):
    bsz, n, _ = input1.shape
    t1 = jnp.transpose(input1, (0, 2, 1))
    t2 = jnp.transpose(input2, (0, 2, 1))
    q = jnp.stack([t1, t2], axis=1).reshape(2 * bsz, 3, n)
    r = jnp.stack([t2, t1], axis=1).reshape(2 * bsz, 3, n)
    ndir = 2 * bsz
    wpd = max(_NW // ndir, 1)
    qpw = n // wpd
    ngrp = qpw // 16

    mesh = plsc.VectorSubcoreMesh(
        core_axis_name="c", subcore_axis_name="s",
        num_cores=_NC, num_subcores=_NS,
    )

    @functools.partial(
        pl.kernel,
        out_type=jax.ShapeDtypeStruct((ndir * n,), jnp.float32),
        mesh=mesh,
        scratch_types=[
            pltpu.VMEM((n,), jnp.float32),
            pltpu.VMEM((n,), jnp.float32),
            pltpu.VMEM((n,), jnp.float32),
            pltpu.VMEM((qpw,), jnp.float32),
            pltpu.VMEM((qpw,), jnp.float32),
            pltpu.VMEM((qpw,), jnp.float32),
            pltpu.VMEM((qpw,), jnp.float32),
        ],
    )
    def sc_kernel(q_hbm, r_hbm, out_hbm, rx, ry, rz, qx, qy, qz, ob):
        wid = lax.axis_index("s") * _NC + lax.axis_index("c")
        d = wid // wpd
        qb = (wid % wpd) * qpw
        rbase = d * 3 * n
        qbase = d * 3 * n + qb
        pltpu.sync_copy(r_hbm.at[pl.ds(rbase, n)], rx)
        pltpu.sync_copy(r_hbm.at[pl.ds(rbase + n, n)], ry)
        pltpu.sync_copy(r_hbm.at[pl.ds(rbase + 2 * n, n)], rz)
        pltpu.sync_copy(q_hbm.at[pl.ds(qbase, qpw)], qx)
        pltpu.sync_copy(q_hbm.at[pl.ds(qbase + n, qpw)], qy)
        pltpu.sync_copy(q_hbm.at[pl.ds(qbase + 2 * n, qpw)], qz)

        def group_body(g, _):
            base = g * 16
            qxv = qx[pl.ds(base, 16)]
            qyv = qy[pl.ds(base, 16)]
            qzv = qz[pl.ds(base, 16)]
            m0 = jnp.full((16,), 3.4e38, jnp.float32)

            def ref_body(c, m):
                rbase = c * 16
                rxv = rx[pl.ds(rbase, 16)]
                ryv = ry[pl.ds(rbase, 16)]
                rzv = rz[pl.ds(rbase, 16)]
                for k in range(16):
                    dx = qxv - jnp.full((16,), rxv[k])
                    dy = qyv - jnp.full((16,), ryv[k])
                    dz = qzv - jnp.full((16,), rzv[k])
                    dd = dx * dx + dy * dy + dz * dz
                    m = jnp.minimum(m, dd)
                return m

            m = lax.fori_loop(0, n // 16, ref_body, m0)
            ob[pl.ds(base, 16)] = m
            return 0

        lax.fori_loop(0, ngrp, group_body, 0)
        pltpu.sync_copy(ob, out_hbm.at[pl.ds(d * n + qb, qpw)])

    out = sc_kernel(q.reshape(-1), r.reshape(-1)).reshape(ndir, n)
    return out[0::2], out[1::2]


def _hi_lo(x):
    hi = x.astype(jnp.bfloat16)
    lo = (x - hi.astype(jnp.float32)).astype(jnp.bfloat16)
    return hi, lo


def _nnd_one(p1, p2, d1_out, d2_out):
    n1 = jnp.sum(p1 * p1, axis=0, keepdims=True)
    n2 = jnp.sum(p2 * p2, axis=0, keepdims=True)
    b2 = -2.0 * p2
    p1h, p1l = _hi_lo(p1)
    b2h, b2l = _hi_lo(b2)
    n1h, n1l = _hi_lo(n1)
    n2h, n2l = _hi_lo(n2)
    one = jnp.ones_like(n1h)
    a = jnp.concatenate([p1h, p1h, p1l, n1h, n1l, one, one], axis=0)
    b = jnp.concatenate([b2h, b2l, b2h, one, one, n2h, n2l], axis=0)
    n = p1.shape[1]
    nh = n // 2
    m1 = None
    for jh in range(2):
        d = jax.lax.dot_general(
            a, b[:, jh * nh:(jh + 1) * nh], (((0,), (0,)), ((), ())),
            preferred_element_type=jnp.float32,
        )
        mh = jnp.min(d, axis=1)
        m1 = mh if m1 is None else jnp.minimum(m1, mh)
        d2_out[pl.ds(jh * nh, nh)] = jnp.min(d, axis=0)
    d1_out[...] = m1


def _nnd_batch_kernel(p1_ref, p2_ref, d1_ref, d2_ref):
    for b in range(p1_ref.shape[0]):
        _nnd_one(p1_ref[b], p2_ref[b], d1_ref.at[b, 0], d2_ref.at[b, 0])


def _nnd_pallas(input1, input2):
    bsz, n, _ = input1.shape
    t1 = jnp.transpose(input1, (0, 2, 1))
    t2 = jnp.transpose(input2, (0, 2, 1))
    bpb = 8 if bsz % 8 == 0 else 1
    grid = (bsz // bpb,)
    out_shape = (
        jax.ShapeDtypeStruct((bsz, 1, n), jnp.float32),
        jax.ShapeDtypeStruct((bsz, 1, n), jnp.float32),
    )
    d1, d2 = pl.pallas_call(
        _nnd_batch_kernel,
        grid=grid,
        in_specs=[
            pl.BlockSpec((bpb, 3, n), lambda b: (b, 0, 0)),
            pl.BlockSpec((bpb, 3, n), lambda b: (b, 0, 0)),
        ],
        out_specs=(
            pl.BlockSpec((bpb, 1, n), lambda b: (b, 0, 0)),
            pl.BlockSpec((bpb, 1, n), lambda b: (b, 0, 0)),
        ),
        out_shape=out_shape,
    )(t1, t2)
    return d1.reshape(bsz, n), d2.reshape(bsz, n)


def kernel(input1, input2):
    return _nnd_pallas(input1, input2)
